# baseline (speedup 1.0000x reference)
import functools

import jax
import jax.numpy as jnp
from jax import lax
from jax.experimental import pallas as pl
from jax.experimental.pallas import tpu as pltpu
from jax.experimental.pallas import tpu_sc as plsc

_B = 4
_S = 2048
_D = 512
_N = _B * _S
_NW = 32
_TOK_PER_W = _N // _NW
_C = 32
_NCH = _TOK_PER_W // _C
_VPC = _C * _D // 16


def _emb_kernel(idx_hbm, word_hbm, pos_hbm, out_hbm,
                idx_v, w0, w1, p0, p1, sg0, sg1, sp0, sp1, so0, so1):
    wid = lax.axis_index("s") * 2 + lax.axis_index("c")
    base = wid * _TOK_PER_W
    pos_base = base % _S
    pltpu.sync_copy(idx_hbm.at[wid], idx_v)
    wbufs, pbufs = (w0, w1), (p0, p1)
    sgs, sps, sos = (sg0, sg1), (sp0, sp1), (so0, so1)

    gs = [pltpu.async_copy(word_hbm.at[idx_v.at[0]], w0, sg0), None]
    ps = [pltpu.async_copy(pos_hbm.at[pl.ds(pos_base, _C)], p0, sp0), None]
    outs = [None, None]
    for c in range(_NCH):
        cur, nxt = c % 2, (c + 1) % 2
        if c + 1 < _NCH:
            if outs[nxt] is not None:
                outs[nxt].wait()
            gs[nxt] = pltpu.async_copy(
                word_hbm.at[idx_v.at[c + 1]], wbufs[nxt], sgs[nxt])
            ps[nxt] = pltpu.async_copy(
                pos_hbm.at[pl.ds(pos_base + (c + 1) * _C, _C)],
                pbufs[nxt], sps[nxt])
        gs[cur].wait()
        ps[cur].wait()
        w, p = wbufs[cur], pbufs[cur]

        def add_body(r, _, w=w, p=p):
            for j in range(_D // 16):
                plsc.addupdate(w.at[r, pl.ds(j * 16, 16)],
                               p[r, pl.ds(j * 16, 16)])
            return 0

        lax.fori_loop(0, _C, add_body, 0)
        outs[cur] = pltpu.async_copy(
            w, out_hbm.at[pl.ds(base + c * _C, _C)], sos[cur])
    outs[0].wait()
    outs[1].wait()


@jax.jit
def _run(idx3d, word_emb, pos_emb):
    mesh = plsc.VectorSubcoreMesh(core_axis_name="c", subcore_axis_name="s")
    k = functools.partial(
        pl.kernel,
        mesh=mesh,
        out_type=jax.ShapeDtypeStruct((_N, _D), jnp.float32),
        scratch_types=[
            pltpu.VMEM((_NCH, _C), jnp.int32),
            pltpu.VMEM((_C, _D), jnp.float32),
            pltpu.VMEM((_C, _D), jnp.float32),
            pltpu.VMEM((_C, _D), jnp.float32),
            pltpu.VMEM((_C, _D), jnp.float32),
            pltpu.SemaphoreType.DMA,
            pltpu.SemaphoreType.DMA,
            pltpu.SemaphoreType.DMA,
            pltpu.SemaphoreType.DMA,
            pltpu.SemaphoreType.DMA,
            pltpu.SemaphoreType.DMA,
        ],
    )(_emb_kernel)
    return k(idx3d, word_emb, pos_emb)


def kernel(inputs, word_emb, pos_emb):
    idx3d = inputs.reshape(_NW, _NCH, _C)
    out = _run(idx3d, word_emb, pos_emb)
    return out.reshape(_B, _S, _D)


# speedup vs baseline: 1.4090x; 1.4090x over previous
import functools

import jax
import jax.numpy as jnp
from jax import lax
from jax.experimental import pallas as pl
from jax.experimental.pallas import tpu as pltpu
from jax.experimental.pallas import tpu_sc as plsc

_B = 4
_S = 2048
_D = 512
_NW = 32
_C = _S // _NW


def _emb_kernel(idx_hbm, word_hbm, pos_hbm, out_hbm,
                idx_v, pos_v, w0, w1, sg0, sg1, so0, so1):
    wid = lax.axis_index("s") * 2 + lax.axis_index("c")
    pos_base = wid * _C
    pltpu.sync_copy(idx_hbm.at[:, wid], idx_v)
    pp = pltpu.async_copy(pos_hbm.at[pl.ds(pos_base, _C)], pos_v, sg1)
    wbufs, sgs, sos = (w0, w1), (sg0, sg1), (so0, so1)

    gs = [pltpu.async_copy(word_hbm.at[idx_v.at[0]], w0, sg0), None]
    outs = [None, None]
    pp.wait()
    for b in range(_B):
        cur, nxt = b % 2, (b + 1) % 2
        if b + 1 < _B:
            if outs[nxt] is not None:
                outs[nxt].wait()
            gs[nxt] = pltpu.async_copy(
                word_hbm.at[idx_v.at[b + 1]], wbufs[nxt], sgs[nxt])
        gs[cur].wait()
        w = wbufs[cur]

        def add_body(r, _, w=w):
            for g in range(_D // 128):
                vals = [pos_v[r, pl.ds((g * 8 + j) * 16, 16)]
                        for j in range(8)]
                for j in range(8):
                    plsc.addupdate(w.at[r, pl.ds((g * 8 + j) * 16, 16)],
                                   vals[j])
            return 0

        lax.fori_loop(0, _C, add_body, 0)
        outs[cur] = pltpu.async_copy(
            w, out_hbm.at[pl.ds(b * _S + pos_base, _C)], sos[cur])
    outs[0].wait()
    outs[1].wait()


@jax.jit
def _run(idx3d, word_emb, pos_emb):
    mesh = plsc.VectorSubcoreMesh(core_axis_name="c", subcore_axis_name="s")
    k = functools.partial(
        pl.kernel,
        mesh=mesh,
        out_type=jax.ShapeDtypeStruct((_B * _S, _D), jnp.float32),
        scratch_types=[
            pltpu.VMEM((_B, _C), jnp.int32),
            pltpu.VMEM((_C, _D), jnp.float32),
            pltpu.VMEM((_C, _D), jnp.float32),
            pltpu.VMEM((_C, _D), jnp.float32),
            pltpu.SemaphoreType.DMA,
            pltpu.SemaphoreType.DMA,
            pltpu.SemaphoreType.DMA,
            pltpu.SemaphoreType.DMA,
        ],
    )(_emb_kernel)
    return k(idx3d, word_emb, pos_emb)


def kernel(inputs, word_emb, pos_emb):
    idx3d = inputs.reshape(_B, _NW, _C)
    out = _run(idx3d, word_emb, pos_emb)
    return out.reshape(_B, _S, _D)
